# baseline (speedup 1.0000x reference)
import functools
import jax
import jax.numpy as jnp
from jax import lax
from jax.experimental import pallas as pl
from jax.experimental.pallas import tpu as pltpu
from jax.experimental.pallas import tpu_sc as plsc

H = 128
N_NODES = 50000
E_EDGES = 800000
_C = 3200
_NCHUNK = E_EDGES // _C
_NW = 32


def _segsum_body(vt_hbm, dst_hbm, out_hbm,
                 d_a, v0_a, v1_a, d_b, v0_b, v1_b,
                 acc0, acc1, sem_a, sem_b):
    wid = lax.axis_index("s") * 2 + lax.axis_index("c")

    def start(chunk, f0, bufs, sem):
        d_buf, v0_buf, v1_buf = bufs
        pltpu.async_copy(dst_hbm.at[pl.ds(chunk * _C, _C)], d_buf, sem)
        pltpu.async_copy(vt_hbm.at[f0, pl.ds(chunk * _C, _C)], v0_buf, sem)
        pltpu.async_copy(vt_hbm.at[f0 + 1, pl.ds(chunk * _C, _C)], v1_buf, sem)

    def wait(bufs, sem):
        d_buf, v0_buf, v1_buf = bufs
        pltpu.make_async_copy(dst_hbm.at[pl.ds(0, _C)], d_buf, sem).wait()
        pltpu.make_async_copy(vt_hbm.at[0, pl.ds(0, _C)], v0_buf, sem).wait()
        pltpu.make_async_copy(vt_hbm.at[0, pl.ds(0, _C)], v1_buf, sem).wait()

    def consume(bufs):
        d_buf, v0_buf, v1_buf = bufs

        def inner(j, _):
            d = d_buf[pl.ds(j * 16, 16)]
            plsc.addupdate_scatter(acc0, [d], v0_buf[pl.ds(j * 16, 16)])
            plsc.addupdate_scatter(acc1, [d], v1_buf[pl.ds(j * 16, 16)])
            return 0

        lax.fori_loop(0, _C // 16, inner, 0)

    bufs_a = (d_a, v0_a, v1_a)
    bufs_b = (d_b, v0_b, v1_b)
    zeros16 = jnp.zeros((16,), jnp.float32)

    for p in range(2):
        f0 = (p * _NW + wid) * 2

        def zero(i, _):
            acc0[pl.ds(i * 16, 16)] = zeros16
            acc1[pl.ds(i * 16, 16)] = zeros16
            return 0

        lax.fori_loop(0, N_NODES // 16, zero, 0)

        start(0, f0, bufs_a, sem_a)

        def body2(h, _):
            c0 = h * 2
            start(c0 + 1, f0, bufs_b, sem_b)
            wait(bufs_a, sem_a)
            consume(bufs_a)

            @pl.when(c0 + 2 < _NCHUNK)
            def _():
                start(c0 + 2, f0, bufs_a, sem_a)

            wait(bufs_b, sem_b)
            consume(bufs_b)
            return 0

        lax.fori_loop(0, _NCHUNK // 2, body2, 0)

        pltpu.sync_copy(acc0, out_hbm.at[f0])
        pltpu.sync_copy(acc1, out_hbm.at[f0 + 1])


@jax.jit
def _sc_segsum_t(vt, dst):
    mesh = plsc.VectorSubcoreMesh(core_axis_name="c", subcore_axis_name="s")
    return pl.kernel(
        _segsum_body,
        mesh=mesh,
        compiler_params=pltpu.CompilerParams(needs_layout_passes=False),
        out_type=jax.ShapeDtypeStruct((H, N_NODES), jnp.float32),
        scratch_types=[
            pltpu.VMEM((_C,), jnp.int32),
            pltpu.VMEM((_C,), jnp.float32),
            pltpu.VMEM((_C,), jnp.float32),
            pltpu.VMEM((_C,), jnp.int32),
            pltpu.VMEM((_C,), jnp.float32),
            pltpu.VMEM((_C,), jnp.float32),
            pltpu.VMEM((N_NODES,), jnp.float32),
            pltpu.VMEM((N_NODES,), jnp.float32),
            pltpu.SemaphoreType.DMA,
            pltpu.SemaphoreType.DMA,
        ],
    )(vt, dst)


def _branch(nf, ef, src, dst, gid, p):
    n = nf.shape[0]
    g = 1024

    wu = p['l1_upd_e'][0]
    w1, w2, w3 = wu[:H], wu[H:2 * H], wu[2 * H:]

    nn1 = jnp.maximum(nf @ p['l1_n2n'][0] + p['l1_n2n'][1], 0.0)
    a_tab = jnp.maximum(nf @ p['l1_left'][0] + p['l1_left'][1], 0.0) @ w1
    b_tab = jnp.maximum(nf @ p['l1_right'][0] + p['l1_right'][1], 0.0) @ w2

    e2n = jnp.maximum(ef @ p['l1_e2n'][0] + p['l1_e2n'][1], 0.0)
    t = jnp.maximum(ef @ p['l1_e2e'][0] + p['l1_e2e'][1], 0.0) @ w3
    new_e = jnp.maximum(a_tab[src] + b_tab[dst] + t + p['l1_upd_e'][1], 0.0)
    e2n2 = jnp.maximum(new_e @ p['l2_e2n'][0] + p['l2_e2n'][1], 0.0)

    agg = _sc_segsum_t(e2n.T, dst).T
    agg2 = _sc_segsum_t(e2n2.T, dst).T

    wun = p['l1_upd_n'][0]
    wun2 = p['l2_upd_n'][0]
    new_n = jnp.maximum(nn1 @ wun[:H] + agg @ wun[H:] + p['l1_upd_n'][1], 0.0)
    nn2 = jnp.maximum(new_n @ p['l2_n2n'][0] + p['l2_n2n'][1], 0.0)
    h = jnp.maximum(nn2 @ wun2[:H] + agg2 @ wun2[H:] + p['l2_upd_n'][1], 0.0)
    hh = jnp.tanh(h @ p['n2g'][0] + p['n2g'][1])

    mu = jnp.mean(hh, axis=0)
    var = jnp.mean(hh * hh, axis=0) - mu * mu
    gamma, beta = p['bn1']
    scale = gamma * jax.lax.rsqrt(var + 1e-5)
    shift = beta - scale * mu

    seg = jax.ops.segment_sum(hh, gid, num_segments=g)
    cnt = jax.ops.segment_sum(jnp.ones((n,), jnp.float32), gid, num_segments=g)
    gfeat = seg * scale + cnt[:, None] * shift

    wp, bp = p['pred']
    return gfeat @ wp + bp


def kernel(node_feats1, edge_feats1, node_feats2, edge_feats2, edge_index1,
           graph_ids1, edge_index2, graph_ids2, params):
    s1 = _branch(node_feats1, edge_feats1, edge_index1[0], edge_index1[1],
                 graph_ids1, params)
    s2 = _branch(node_feats2, edge_feats2, edge_index2[0], edge_index2[1],
                 graph_ids2, params)
    diff = s1 - s2
    wf, bf = params['fc']
    x = diff @ wf + bf
    g2, b2 = params['bn2']
    mu = jnp.mean(x, axis=0)
    var = jnp.mean((x - mu) ** 2, axis=0)
    x = jnp.maximum(g2 * (x - mu) * jax.lax.rsqrt(var + 1e-5) + b2, 0.0)
    wo, bo = params['out']
    return jnp.squeeze(x @ wo + bo, axis=-1)
